# baseline (speedup 1.0000x reference)
import functools

import jax
import jax.numpy as jnp
from jax import lax
from jax.experimental import pallas as pl
from jax.experimental.pallas import tpu as pltpu
from jax.experimental.pallas import tpu_sc as plsc


def _sc_kernel(B, D, L, NC, BPW, GCHUNK):
    mesh = plsc.VectorSubcoreMesh(core_axis_name="c", subcore_axis_name="s")
    n_chunks = BPW // GCHUNK
    groups_per_chunk = GCHUNK // L

    @functools.partial(
        pl.kernel,
        mesh=mesh,
        out_type=jax.ShapeDtypeStruct((B,), jnp.float32),
        scratch_types=[
            pltpu.VMEM((BPW,), jnp.int32),
            pltpu.VMEM((BPW, D), jnp.float32),
            pltpu.VMEM((D + 8,), jnp.float32),
            pltpu.VMEM((BPW,), jnp.float32),
            pltpu.SemaphoreType.DMA,
        ],
        compiler_params=pltpu.CompilerParams(needs_layout_passes=False),
    )
    def k(idx_hbm, table_hbm, wb_hbm, out_hbm,
          idx_v, rows_v, wb_v, out_v, sem):
        wid = lax.axis_index("s") * NC + lax.axis_index("c")
        base = wid * BPW

        idx_copy = pltpu.async_copy(idx_hbm.at[pl.ds(base, BPW)], idx_v, sem)
        pltpu.sync_copy(wb_hbm, wb_v)
        idx_copy.wait()

        copies = []
        for g in range(n_chunks):
            copies.append(pltpu.async_copy(
                table_hbm.at[idx_v.at[pl.ds(g * GCHUNK, GCHUNK)]],
                rows_v.at[pl.ds(g * GCHUNK, GCHUNK), :],
                sem,
            ))

        w_chunks = [wb_v[pl.ds(c * L, L)] for c in range(D // L)]
        lane = lax.iota(jnp.int32, L)
        bscalar = plsc.load_gather(wb_v, [jnp.zeros_like(lane) + D])
        m8 = lane < (L // 2)
        xors = [lane ^ s for s in (8, 4, 2, 1)]
        half = lane & (L // 2 - 1)
        packs = [
            (half & 3) | ((half & 4) << 1),
            (half & 1) | ((half >> 1) << 2),
            half << 1,
        ]

        def g16(v, p):
            return v.at[p].get(mode="promise_in_bounds")

        def combine(level, a, b):
            ta = a + g16(a, xors[level])
            tb = b + g16(b, xors[level])
            if level == 0:
                return jnp.where(m8, ta, tb)
            p = packs[level - 1]
            return jnp.where(m8, g16(ta, p), g16(tb, p))

        for c in copies:
            c.wait()

        @plsc.parallel_loop(0, n_chunks * groups_per_chunk, unroll=4)
        def group_body(j):
            j16 = j * L
            stack = []
            for r in range(L):
                s = rows_v[j16 + r, pl.ds(0, L)] * w_chunks[0]
                for c in range(1, D // L):
                    s = rows_v[j16 + r, pl.ds(c * L, L)] * w_chunks[c] + s
                stack.append((0, s))
                while len(stack) >= 2 and stack[-1][0] == stack[-2][0]:
                    lv, bb = stack.pop()
                    _, aa = stack.pop()
                    stack.append((lv + 1, combine(lv, aa, bb)))
            logit = stack[0][1] + bscalar
            out_v[pl.ds(j16, L)] = 1.0 / (1.0 + jnp.exp(-logit))

        pltpu.sync_copy(out_v, out_hbm.at[pl.ds(base, BPW)])

    return k


def kernel(item_indices, embedding_table, pred_W, pred_b):
    B = item_indices.shape[0]
    V, D = embedding_table.shape
    info = plsc.get_sparse_core_info()
    NC, NS, L = info.num_cores, info.num_subcores, info.num_lanes
    NW = NC * NS
    BPW = B // NW
    GCHUNK = 256

    w_flat = pred_W.reshape(D).astype(jnp.float32)
    wb = jnp.concatenate(
        [w_flat, pred_b.astype(jnp.float32),
         jnp.zeros((7,), jnp.float32)])

    out = _sc_kernel(B, D, L, NC, BPW, GCHUNK)(
        item_indices.astype(jnp.int32), embedding_table, wb)
    return out.reshape(B, 1)


# speedup vs baseline: 1.1079x; 1.1079x over previous
import functools

import jax
import jax.numpy as jnp
from jax import lax
from jax.experimental import pallas as pl
from jax.experimental.pallas import tpu as pltpu
from jax.experimental.pallas import tpu_sc as plsc


def _sc_kernel(B, D, L, NC, BPW, GCHUNK):
    mesh = plsc.VectorSubcoreMesh(core_axis_name="c", subcore_axis_name="s")
    n_chunks = BPW // GCHUNK
    groups_per_chunk = GCHUNK // L

    @functools.partial(
        pl.kernel,
        mesh=mesh,
        out_type=jax.ShapeDtypeStruct((B,), jnp.float32),
        scratch_types=[
            pltpu.VMEM((BPW,), jnp.int32),
            pltpu.VMEM((BPW, D), jnp.float32),
            pltpu.VMEM((D + 8,), jnp.float32),
            pltpu.VMEM((BPW,), jnp.float32),
            pltpu.SemaphoreType.DMA,
        ],
        compiler_params=pltpu.CompilerParams(needs_layout_passes=False),
    )
    def k(idx_hbm, table_hbm, wb_hbm, out_hbm,
          idx_v, rows_v, wb_v, out_v, sem):
        wid = lax.axis_index("s") * NC + lax.axis_index("c")
        base = wid * BPW

        idx_copy = pltpu.async_copy(idx_hbm.at[pl.ds(base, BPW)], idx_v, sem)
        pltpu.sync_copy(wb_hbm, wb_v)
        idx_copy.wait()

        copies = []
        for g in range(n_chunks):
            copies.append(pltpu.async_copy(
                table_hbm.at[idx_v.at[pl.ds(g * GCHUNK, GCHUNK)]],
                rows_v.at[pl.ds(g * GCHUNK, GCHUNK), :],
                sem,
            ))

        w_chunks = [wb_v[pl.ds(c * L, L)] for c in range(D // L)]
        lane = lax.iota(jnp.int32, L)
        bscalar = plsc.load_gather(wb_v, [jnp.zeros_like(lane) + D])
        m8 = lane < (L // 2)
        xors = [lane ^ s for s in (8, 4, 2, 1)]
        half = lane & (L // 2 - 1)
        packs = [
            (half & 3) | ((half & 4) << 1),
            (half & 1) | ((half >> 1) << 2),
            half << 1,
        ]

        def g16(v, p):
            return v.at[p].get(mode="promise_in_bounds")

        def combine(level, a, b):
            ta = a + g16(a, xors[level])
            tb = b + g16(b, xors[level])
            if level == 0:
                return jnp.where(m8, ta, tb)
            p = packs[level - 1]
            return jnp.where(m8, g16(ta, p), g16(tb, p))

        for c in copies:
            c.wait()

        @plsc.parallel_loop(0, n_chunks * groups_per_chunk, unroll=2)
        def group_body(j):
            j16 = j * L
            stack = []
            for r in range(L):
                s = rows_v[j16 + r, pl.ds(0, L)] * w_chunks[0]
                for c in range(1, D // L):
                    s = rows_v[j16 + r, pl.ds(c * L, L)] * w_chunks[c] + s
                stack.append((0, s))
                while len(stack) >= 2 and stack[-1][0] == stack[-2][0]:
                    lv, bb = stack.pop()
                    _, aa = stack.pop()
                    stack.append((lv + 1, combine(lv, aa, bb)))
            logit = stack[0][1] + bscalar
            out_v[pl.ds(j16, L)] = 1.0 / (1.0 + jnp.exp(-logit))

        pltpu.sync_copy(out_v, out_hbm.at[pl.ds(base, BPW)])

    return k


def kernel(item_indices, embedding_table, pred_W, pred_b):
    B = item_indices.shape[0]
    V, D = embedding_table.shape
    info = plsc.get_sparse_core_info()
    NC, NS, L = info.num_cores, info.num_subcores, info.num_lanes
    NW = NC * NS
    BPW = B // NW
    GCHUNK = 256

    w_flat = pred_W.reshape(D).astype(jnp.float32)
    wb = jnp.concatenate(
        [w_flat, pred_b.astype(jnp.float32),
         jnp.zeros((7,), jnp.float32)])

    out = _sc_kernel(B, D, L, NC, BPW, GCHUNK)(
        item_indices.astype(jnp.int32), embedding_table, wb)
    return out.reshape(B, 1)
